# baseline (speedup 1.0000x reference)
import jax
from jax.experimental import pallas as pl

_GRID = 10


def _copy_body(x_ref, e_ref, xo_ref, eo_ref):
    xo_ref[...] = x_ref[...]
    eo_ref[...] = e_ref[...]


def kernel(x, edge_index, edge_attr):
    del edge_index
    n_nodes, d_feat = x.shape
    n_edges, d_edge = edge_attr.shape
    e2 = edge_attr.reshape(n_edges * d_edge // 128, 128)
    bx = n_nodes // _GRID
    be = e2.shape[0] // _GRID
    x_out, e_out = pl.pallas_call(
        _copy_body,
        grid=(_GRID,),
        in_specs=[
            pl.BlockSpec((bx, d_feat), lambda i: (i, 0)),
            pl.BlockSpec((be, 128), lambda i: (i, 0)),
        ],
        out_specs=[
            pl.BlockSpec((bx, d_feat), lambda i: (i, 0)),
            pl.BlockSpec((be, 128), lambda i: (i, 0)),
        ],
        out_shape=[
            jax.ShapeDtypeStruct(x.shape, x.dtype),
            jax.ShapeDtypeStruct(e2.shape, e2.dtype),
        ],
    )(x, e2)
    return (x_out, e_out.reshape(n_edges, d_edge))


# speedup vs baseline: 1.1026x; 1.1026x over previous
import jax
from jax.experimental import pallas as pl

_GRID = 25


def _copy_body(x_ref, e_ref, xo_ref, eo_ref):
    xo_ref[...] = x_ref[...]
    eo_ref[...] = e_ref[...]


def kernel(x, edge_index, edge_attr):
    del edge_index
    n_nodes, d_feat = x.shape
    n_edges, d_edge = edge_attr.shape
    bx = n_nodes // _GRID
    be = n_edges // _GRID
    x_out, e_out = pl.pallas_call(
        _copy_body,
        grid=(_GRID,),
        in_specs=[
            pl.BlockSpec((bx, d_feat), lambda i: (i, 0)),
            pl.BlockSpec((be, d_edge), lambda i: (i, 0)),
        ],
        out_specs=[
            pl.BlockSpec((bx, d_feat), lambda i: (i, 0)),
            pl.BlockSpec((be, d_edge), lambda i: (i, 0)),
        ],
        out_shape=[
            jax.ShapeDtypeStruct(x.shape, x.dtype),
            jax.ShapeDtypeStruct(edge_attr.shape, edge_attr.dtype),
        ],
    )(x, edge_attr)
    return (x_out, e_out)
